# baseline (speedup 1.0000x reference)
import functools

import jax
import jax.numpy as jnp
from jax import lax
from jax.experimental import pallas as pl
from jax.experimental.pallas import tpu as pltpu
from jax.experimental.pallas import tpu_sc as plsc

N_NODES = 100000
D = 128
B = 10000
S = 16
L = 16
NC, NS = 2, 16
NW = NC * NS
CN = 8
NCHUNKS = B // CN


def _body(table_hbm, idx_hbm, out_hbm, idx_v, rows_v, out_v, sem):
    wid = lax.axis_index("s") * NC + lax.axis_index("c")
    nk = (NCHUNKS - wid + (NW - 1)) // NW

    def chunk_step(k, carry):
        c = wid + k * NW
        pltpu.sync_copy(idx_hbm.at[pl.ds(c * CN * S, CN * S)], idx_v)
        pltpu.async_copy(table_hbm.at[idx_v], rows_v, sem).wait()
        inv = jnp.full((L,), 1.0 / S, dtype=jnp.float32)
        for i in range(CN):
            for j in range(D // L):
                acc = rows_v[i * S, pl.ds(j * L, L)]
                for s in range(1, S):
                    acc = acc + rows_v[i * S + s, pl.ds(j * L, L)]
                out_v[i, pl.ds(j * L, L)] = acc * inv
        pltpu.sync_copy(out_v, out_hbm.at[pl.ds(c * CN, CN)])
        return carry

    lax.fori_loop(0, nk, chunk_step, 0)


@jax.jit
def _sc_mean_agg(table, idx_flat):
    mesh = plsc.VectorSubcoreMesh(core_axis_name="c", subcore_axis_name="s")
    kfn = pl.kernel(
        _body,
        mesh=mesh,
        out_type=jax.ShapeDtypeStruct((B, D), jnp.float32),
        scratch_types=[
            pltpu.VMEM((CN * S,), jnp.int32),
            pltpu.VMEM((CN * S, D), jnp.float32),
            pltpu.VMEM((CN, D), jnp.float32),
            pltpu.SemaphoreType.DMA,
        ],
    )
    return kfn(table, idx_flat)


def kernel(features_weight, nodes, neigh_idx):
    idx_flat = neigh_idx.astype(jnp.int32).reshape(B * S)
    return _sc_mean_agg(features_weight, idx_flat)
